# baseline (speedup 1.0000x reference)
import jax
import jax.numpy as jnp
from jax import lax
from jax.experimental import pallas as pl
from jax.experimental.pallas import tpu as pltpu
from jax.experimental.pallas import tpu_sc as plsc

_J = 17
_HW = 25
_N = _HW * _J


def _scs_body(x_hbm, out_hbm, x_s, o_s):
    pltpu.sync_copy(x_hbm, x_s)

    def row(r, carry):
        base = r * _J
        best = x_s[base]
        idx = jnp.int32(0)
        for j in range(1, _J):
            v = x_s[base + j]
            gt = v > best
            best = jnp.where(gt, v, best)
            idx = jnp.where(gt, jnp.int32(j), idx)
        for j in range(_J):
            v = x_s[base + j]
            y = jnp.where(idx == j, jnp.float32(1.0), jnp.float32(0.0))
            o_s[base + j] = (y - v) + v
        return carry

    lax.fori_loop(0, _HW, row, jnp.int32(0))
    pltpu.sync_copy(o_s, out_hbm)


def kernel(sgt_trans_mat, use_gumbel_noise, gumbel_temp):
    mesh = plsc.ScalarSubcoreMesh(axis_name="c", num_cores=1)
    flat = pl.kernel(
        _scs_body,
        out_type=jax.ShapeDtypeStruct((_N,), jnp.float32),
        mesh=mesh,
        scratch_types=[
            pltpu.SMEM((_N,), jnp.float32),
            pltpu.SMEM((_N,), jnp.float32),
        ],
    )(sgt_trans_mat.reshape(_N))
    return flat.reshape(1, _HW, _J)
